# baseline (speedup 1.0000x reference)
import jax
import jax.numpy as jnp
from jax.experimental import pallas as pl

_B = 4
_N = 2048
_K = 9


def _gram_body(a_ref, b_ref, o_ref):
    o_ref[0] = jnp.dot(a_ref[0], b_ref[0],
                       preferred_element_type=jnp.float32)


def _batched_gram(f, ft):
    b, n, c = f.shape
    return pl.pallas_call(
        _gram_body,
        grid=(b,),
        in_specs=[pl.BlockSpec((1, n, c), lambda i: (i, 0, 0)),
                  pl.BlockSpec((1, c, n), lambda i: (i, 0, 0))],
        out_specs=pl.BlockSpec((1, n, n), lambda i: (i, 0, 0)),
        out_shape=jax.ShapeDtypeStruct((b, n, n), jnp.float32))(f, ft)


def _bnorm(x, g, b, eps=1e-5):
    m = jnp.mean(x, axis=(0, 2, 3), keepdims=True)
    v = jnp.var(x, axis=(0, 2, 3), keepdims=True)
    xn = (x - m) / jnp.sqrt(v + eps)
    return xn * g[None, :, None, None] + b[None, :, None, None]


def _conv1x1(x, w):
    return jnp.einsum("oc,bcnk->bonk", w, x)


def _activate(x, a):
    if a == "relu":
        return jax.nn.relu(x)
    if a == "leaky":
        return jax.nn.leaky_relu(x, 0.2)
    return x


def _basic(x, p, act, norm=True):
    y = _conv1x1(x, p["W"])
    if "b" in p:
        y = y + p["b"][None, :, None, None]
    if norm:
        y = _bnorm(y, p["gamma"], p["beta"])
    return _activate(y, act)


def _edge(xc, nn_idx, p):
    f = xc[:, :, :, 0]
    k = nn_idx.shape[-1]
    idxb = jnp.broadcast_to(nn_idx.reshape(_B, 1, _N * k), (_B, 32, _N * k))
    x_j = jnp.take_along_axis(f, idxb, axis=2).reshape(_B, 32, _N, k)
    x_i = jnp.broadcast_to(f[:, :, :, None], (_B, 32, _N, k))
    feat = jnp.concatenate([x_i, x_j - x_i], axis=1)
    y = _conv1x1(feat, p["W"])
    y = _bnorm(y, p["gamma"], p["beta"])
    y = jax.nn.leaky_relu(y, 0.2)
    return jnp.max(y, axis=3, keepdims=True)


def _cell(s0, s1, nn_idx, cp):
    s0 = _basic(s0, cp["pre0"], "relu")
    s1 = _basic(s1, cp["pre1"], "relu")
    states = [s0, s1]
    off = 0
    for _ in range(4):
        new = sum(_edge(h, nn_idx, cp["ops"][off + j])
                  for j, h in enumerate(states))
        off += len(states)
        states.append(new)
    return jnp.concatenate(states[-4:], axis=1)


def kernel(x, params):
    s0 = _basic(x, params["stem"], None)

    f = s0[:, :, :, 0].transpose(0, 2, 1)
    sq = jnp.sum(f * f, axis=-1)
    e = _batched_gram(f, f.transpose(0, 2, 1))
    dist = sq[:, :, None] - 2.0 * e + sq[:, None, :]
    _, idx9 = jax.lax.top_k(-dist, _K)
    _, idx18 = jax.lax.top_k(-dist, 2 * _K)
    idxd2 = idx18[:, :, ::2]

    s1 = s0
    c0 = _cell(s0, s1, idx9, params["cells"][0])
    c1 = _cell(s1, c0, idxd2, params["cells"][1])

    fusion = jnp.concatenate([s1, c0, c1], axis=1)
    fusion = _basic(fusion, params["fusion"], "leaky")
    x1 = jnp.max(fusion, axis=(2, 3), keepdims=True)
    x2 = jnp.mean(fusion, axis=(2, 3), keepdims=True)
    h = jnp.concatenate([x1, x2], axis=1)
    h = _basic(h, params["cls1"], "leaky")
    h = _basic(h, params["cls2"], "leaky")
    h = _basic(h, params["cls3"], None, norm=False)
    return h[:, :, 0, 0]


# speedup vs baseline: 19.4617x; 19.4617x over previous
import jax
import jax.numpy as jnp
from jax.experimental import pallas as pl

_B = 4
_N = 2048
_K = 9


def _gram_body(a_ref, b_ref, o_ref):
    o_ref[0] = jnp.dot(a_ref[0], b_ref[0],
                       preferred_element_type=jnp.float32)


def _batched_gram(f, ft):
    b, n, c = f.shape
    return pl.pallas_call(
        _gram_body,
        grid=(b,),
        in_specs=[pl.BlockSpec((1, n, c), lambda i: (i, 0, 0)),
                  pl.BlockSpec((1, c, n), lambda i: (i, 0, 0))],
        out_specs=pl.BlockSpec((1, n, n), lambda i: (i, 0, 0)),
        out_shape=jax.ShapeDtypeStruct((b, n, n), jnp.float32))(f, ft)


def _bnorm(x, g, b, eps=1e-5):
    m = jnp.mean(x, axis=(0, 2, 3), keepdims=True)
    v = jnp.var(x, axis=(0, 2, 3), keepdims=True)
    xn = (x - m) / jnp.sqrt(v + eps)
    return xn * g[None, :, None, None] + b[None, :, None, None]


def _conv1x1(x, w):
    return jnp.einsum("oc,bcnk->bonk", w, x)


def _activate(x, a):
    if a == "relu":
        return jax.nn.relu(x)
    if a == "leaky":
        return jax.nn.leaky_relu(x, 0.2)
    return x


def _basic(x, p, act, norm=True):
    y = _conv1x1(x, p["W"])
    if "b" in p:
        y = y + p["b"][None, :, None, None]
    if norm:
        y = _bnorm(y, p["gamma"], p["beta"])
    return _activate(y, act)


def _edge(xc, nn_idx, p):
    f = xc[:, :, :, 0]
    k = nn_idx.shape[-1]
    ft = f.transpose(0, 2, 1)
    x_jt = ft[jnp.arange(_B)[:, None, None], nn_idx, :]
    x_j = x_jt.transpose(0, 3, 1, 2)
    x_i = jnp.broadcast_to(f[:, :, :, None], (_B, 32, _N, k))
    feat = jnp.concatenate([x_i, x_j - x_i], axis=1)
    y = _conv1x1(feat, p["W"])
    y = _bnorm(y, p["gamma"], p["beta"])
    y = jax.nn.leaky_relu(y, 0.2)
    return jnp.max(y, axis=3, keepdims=True)


def _cell(s0, s1, nn_idx, cp):
    s0 = _basic(s0, cp["pre0"], "relu")
    s1 = _basic(s1, cp["pre1"], "relu")
    states = [s0, s1]
    off = 0
    for _ in range(4):
        new = sum(_edge(h, nn_idx, cp["ops"][off + j])
                  for j, h in enumerate(states))
        off += len(states)
        states.append(new)
    return jnp.concatenate(states[-4:], axis=1)


def kernel(x, params):
    s0 = _basic(x, params["stem"], None)

    f = s0[:, :, :, 0].transpose(0, 2, 1)
    sq = jnp.sum(f * f, axis=-1)
    e = _batched_gram(f, f.transpose(0, 2, 1))
    dist = sq[:, :, None] - 2.0 * e + sq[:, None, :]
    _, idx9 = jax.lax.top_k(-dist, _K)
    _, idx18 = jax.lax.top_k(-dist, 2 * _K)
    idxd2 = idx18[:, :, ::2]

    s1 = s0
    c0 = _cell(s0, s1, idx9, params["cells"][0])
    c1 = _cell(s1, c0, idxd2, params["cells"][1])

    fusion = jnp.concatenate([s1, c0, c1], axis=1)
    fusion = _basic(fusion, params["fusion"], "leaky")
    x1 = jnp.max(fusion, axis=(2, 3), keepdims=True)
    x2 = jnp.mean(fusion, axis=(2, 3), keepdims=True)
    h = jnp.concatenate([x1, x2], axis=1)
    h = _basic(h, params["cls1"], "leaky")
    h = _basic(h, params["cls2"], "leaky")
    h = _basic(h, params["cls3"], None, norm=False)
    return h[:, :, 0, 0]
